# baseline (speedup 1.0000x reference)
import functools

import jax
import jax.numpy as jnp
from jax import lax
from jax.experimental import pallas as pl
from jax.experimental.pallas import tpu as pltpu
from jax.experimental.pallas import tpu_sc as plsc

_V = 100000
_B = 16384
_D = 64
_NC = 2
_NS = 16
_L = 16
_NW = _NC * _NS
_BPW = _B // _NW
_NB = 256
_NBLK = _BPW // _NB

_COPY_ROWS = 2000


def _copy_body(x_ref, o_ref):
    o_ref[...] = x_ref[...]


def _tc_copy(centers):
    return pl.pallas_call(
        _copy_body,
        grid=(_V // _COPY_ROWS,),
        in_specs=[pl.BlockSpec((_COPY_ROWS, _D), lambda i: (i, 0))],
        out_specs=pl.BlockSpec((_COPY_ROWS, _D), lambda i: (i, 0)),
        out_shape=jax.ShapeDtypeStruct((_V, _D), jnp.float32),
    )(centers)


def _sc_partials(features, labels, centers):
    mesh = plsc.VectorSubcoreMesh(core_axis_name="c", subcore_axis_name="s")

    @functools.partial(
        pl.kernel,
        mesh=mesh,
        out_type=jax.ShapeDtypeStruct((_NW, _L), jnp.float32),
        scratch_types=[
            pltpu.VMEM((_BPW,), jnp.int32),
            pltpu.VMEM((_NB, _D), jnp.float32),
            pltpu.VMEM((_NB, _D), jnp.float32),
            pltpu.VMEM((_L,), jnp.float32),
            pltpu.SemaphoreType.DMA,
            pltpu.SemaphoreType.DMA,
        ],
    )
    def k(feat_hbm, idx_hbm, tab_hbm, out_hbm, idx_v, rows_v, feat_v,
          acc_v, sem_g, sem_f):
        wid = lax.axis_index("s") * _NC + lax.axis_index("c")
        base = wid * _BPW
        pltpu.sync_copy(idx_hbm.at[pl.ds(base, _BPW)], idx_v)

        def body(r, accs):
            new = []
            for j in range(_D // _L):
                f = feat_v[r, pl.ds(j * _L, _L)]
                c = rows_v[r, pl.ds(j * _L, _L)]
                d = f - c
                new.append(accs[j] + d * d)
            return tuple(new)

        zero = jnp.zeros((_L,), jnp.float32)
        accs = (zero,) * (_D // _L)
        for b in range(_NBLK):
            feat_cp = pltpu.async_copy(
                feat_hbm.at[pl.ds(base + b * _NB, _NB)], feat_v, sem_f)

            @pl.loop(0, _NB // _L)
            def _(g):
                lv = idx_v[pl.ds(b * _NB + g * _L, _L)]
                for j in range(_L):
                    pltpu.async_copy(tab_hbm.at[pl.ds(lv[j], 1)],
                                     rows_v.at[pl.ds(g * _L + j, 1)], sem_g)

            pltpu.make_async_copy(tab_hbm.at[pl.ds(0, _NB)], rows_v,
                                  sem_g).wait()
            feat_cp.wait()
            accs = lax.fori_loop(0, _NB, body, accs)

        inv_n = 1.0 / (_B * _D)
        acc_v[...] = (accs[0] + accs[1] + accs[2] + accs[3]) * inv_n
        pltpu.sync_copy(acc_v, out_hbm.at[wid])

    return k(features, labels, centers)


def kernel(features, labels, centers):
    labels = labels.astype(jnp.int32)
    tab = _tc_copy(centers)
    partials = _sc_partials(features, labels, tab)
    return jnp.sum(partials)


# speedup vs baseline: 1.6954x; 1.6954x over previous
import functools

import jax
import jax.numpy as jnp
from jax import lax
from jax.experimental import pallas as pl
from jax.experimental.pallas import tpu as pltpu
from jax.experimental.pallas import tpu_sc as plsc

_B = 16384
_D = 64
_NC = 2
_NS = 16
_L = 16
_NW = _NC * _NS
_BPW = _B // _NW
_NB = 256
_NBLK = _BPW // _NB


def _sc_partials(features, labels, centers):
    mesh = plsc.VectorSubcoreMesh(core_axis_name="c", subcore_axis_name="s")

    @functools.partial(
        pl.kernel,
        mesh=mesh,
        compiler_params=pltpu.CompilerParams(skip_device_barrier=True),
        out_type=jax.ShapeDtypeStruct((_NW, _L), jnp.float32),
        scratch_types=[
            pltpu.VMEM((_BPW,), jnp.int32),
            pltpu.VMEM((_NB, _D), jnp.float32),
            pltpu.VMEM((_NB, _D), jnp.float32),
            pltpu.VMEM((_NB, _D), jnp.float32),
            pltpu.VMEM((_L,), jnp.float32),
            pltpu.SemaphoreType.DMA,
            pltpu.SemaphoreType.DMA,
            pltpu.SemaphoreType.DMA,
        ],
    )
    def k(feat_hbm, idx_hbm, tab_hbm, out_hbm, idx_v, rows_a, rows_b,
          feat_v, acc_v, sem_ga, sem_gb, sem_f):
        wid = lax.axis_index("s") * _NC + lax.axis_index("c")
        base = wid * _BPW
        pltpu.sync_copy(idx_hbm.at[pl.ds(base, _BPW)], idx_v)

        row_bufs = (rows_a, rows_b)
        row_sems = (sem_ga, sem_gb)

        def issue_rows(b):
            dst = row_bufs[b % 2]
            sem = row_sems[b % 2]

            @pl.loop(0, _NB // _L)
            def _(g):
                lv = idx_v[pl.ds(b * _NB + g * _L, _L)]
                for j in range(_L):
                    pltpu.async_copy(tab_hbm.at[pl.ds(lv[j], 1)],
                                     dst.at[pl.ds(g * _L + j, 1)], sem)

        issue_rows(0)

        zero = jnp.zeros((_L,), jnp.float32)
        accs = (zero,) * (_D // _L)
        for b in range(_NBLK):
            rows_v = row_bufs[b % 2]

            def body(r, accs, rows_v=rows_v):
                new = []
                for j in range(_D // _L):
                    f = feat_v[r, pl.ds(j * _L, _L)]
                    c = rows_v[r, pl.ds(j * _L, _L)]
                    d = f - c
                    new.append(accs[j] + d * d)
                return tuple(new)

            feat_cp = pltpu.async_copy(
                feat_hbm.at[pl.ds(base + b * _NB, _NB)], feat_v, sem_f)
            if b + 1 < _NBLK:
                issue_rows(b + 1)
            pltpu.make_async_copy(tab_hbm.at[pl.ds(0, _NB)], rows_v,
                                  row_sems[b % 2]).wait()
            feat_cp.wait()
            accs = lax.fori_loop(0, _NB, body, accs)

        inv_n = 1.0 / (_B * _D)
        acc_v[...] = (accs[0] + accs[1] + accs[2] + accs[3]) * inv_n
        pltpu.sync_copy(acc_v, out_hbm.at[wid])

    return k(features, labels, centers)


def kernel(features, labels, centers):
    labels = labels.astype(jnp.int32)
    partials = _sc_partials(features, labels, centers)
    return jnp.sum(partials)
